# baseline (speedup 1.0000x reference)
import functools

import jax
import jax.numpy as jnp
from jax import lax
from jax.experimental import pallas as pl
from jax.experimental.pallas import tpu as pltpu
from jax.experimental.pallas import tpu_sc as plsc

_H = 256
_N = 512
_D = 512
_T = 0.6
_EPS = 1e-6
_MARGIN = 1.0
_LAMBDA_C = 0.5
_BIG = 3.4e38

_NW = 32
_RPW = _N // _NW
_L = 16
_CHUNKS = _N // _L


def _dot_t(a, b):
    return jax.lax.dot_general(a, b, (((1,), (1,)), ((), ())),
                               preferred_element_type=jnp.float32)


def _rowsum(a, ones):
    return jax.lax.dot_general(a, ones, (((1,), (0,)), ((), ())),
                               preferred_element_type=jnp.float32)


def _stage1_kernel(img_ref, cap_ref, hard_ref, mb_ref, sim_tl_ref,
                   sim_tr_ref, aux_ref):
    xi = img_ref[...]
    xj = cap_ref[...]
    ones = jnp.full((_D, 1), 1.0, jnp.float32)
    ss_i = _rowsum(xi * xi, ones)
    ss_j = _rowsum(xj * xj, ones)
    sx_i = _rowsum(xi, ones)
    sx_j = _rowsum(xj, ones)
    ri = 1.0 / jnp.maximum(jnp.sqrt(ss_i), 1e-12)
    rj = 1.0 / jnp.maximum(jnp.sqrt(ss_j), 1e-12)
    zi = xi * ri
    zj = xj * rj

    sim_tl = _dot_t(zi, zi)
    sim_tr = _dot_t(zi, zj)
    sim_bb = _dot_t(zj, zj)

    i2 = jax.lax.broadcasted_iota(jnp.int32, (_H, _H), 0)
    j2 = jax.lax.broadcasted_iota(jnp.int32, (_H, _H), 1)
    diag = i2 == j2
    hard_tl = jnp.where(diag, 0.0, sim_tl)
    hard_tr = jnp.where(diag, 0.0, sim_tr)
    hard_bb = jnp.where(diag, 0.0, sim_bb)

    hard_ref[0:_H, 0:_H] = hard_tl
    hard_ref[0:_H, _H:_N] = hard_tr
    hard_ref[_H:_N, 0:_H] = hard_tr.T
    hard_ref[_H:_N, _H:_N] = hard_bb
    sim_tl_ref[...] = sim_tl
    sim_tr_ref[...] = sim_tr

    hard_bl = hard_tr.T
    m_top = jnp.maximum(jnp.max(hard_tl, axis=1, keepdims=True),
                        jnp.max(hard_tr, axis=1, keepdims=True))
    m_bot = jnp.maximum(jnp.max(hard_bl, axis=1, keepdims=True),
                        jnp.max(hard_bb, axis=1, keepdims=True))
    mb_ref[0:_H, :] = jnp.broadcast_to(m_top, (_H, 128))
    mb_ref[_H:_N, :] = jnp.broadcast_to(m_bot, (_H, 128))

    pos_sel = ((j2 == i2 + 128) & (i2 < 128)) | ((j2 == i2 - 128) & (i2 >= 128))
    sum_pos = jnp.sum(jnp.where(pos_sel, sim_tl, 0.0), keepdims=True)
    contrastive = (jnp.sum(m_top, keepdims=True) + jnp.sum(m_bot, keepdims=True)
                   - 2.0 * sum_pos) / (_T * _H)

    sq_i = ss_i * ri * ri
    sq_j = ss_j * rj * rj
    s_i = sx_i * ri
    s_j = sx_j * rj
    a_term = sq_i + 2.0 * _EPS * s_i + (_D * _EPS * _EPS)
    b_l = sq_i - 2.0 * _EPS * s_i
    b_r = sq_j - 2.0 * _EPS * s_j
    dpos = zi - zj + _EPS
    pos_dist = jnp.sqrt(_rowsum(dpos * dpos, ones))

    aux_ref[0:1, :] = a_term.T
    aux_ref[1:2, :] = b_l.T
    aux_ref[2:3, :] = b_r.T
    aux_ref[3:4, :] = pos_dist.T
    aux_ref[4:5, :] = jnp.broadcast_to(contrastive, (1, _H))


def _sc_stage(hard_hbm, mb_hbm, mem_hbm, rows_v, mrow_v, memacc_v, sem):
    c_ax = lax.axis_index("c")
    s_ax = lax.axis_index("s")
    w = s_ax * 2 + c_ax
    base = w * _RPW
    pltpu.sync_copy(hard_hbm.at[pl.ds(base, _RPW)], rows_v)
    pltpu.sync_copy(mb_hbm.at[pl.ds(base, _RPW)], mrow_v)

    for chunk in range(_CHUNKS):
        memacc_v[pl.ds(chunk * _L, _L)] = jnp.zeros((_L,), jnp.int32)

    def row_body(r, _):
        mb = mrow_v[r, pl.ds(0, _L)]
        for chunk in range(_CHUNKS):
            hit = rows_v[r, pl.ds(chunk * _L, _L)] == mb
            sl = pl.ds(chunk * _L, _L)
            memacc_v[sl] = memacc_v[sl] | jnp.where(hit, 1, 0)
        return 0

    lax.fori_loop(0, _RPW, row_body, 0)

    pltpu.sync_copy(memacc_v, mem_hbm.at[pl.ds(w * _N, _N)])


def _sc_call():
    return pl.kernel(
        _sc_stage,
        mesh=plsc.VectorSubcoreMesh(core_axis_name="c", subcore_axis_name="s"),
        out_type=jax.ShapeDtypeStruct((_NW * _N,), jnp.int32),
        scratch_types=[
            pltpu.VMEM((_RPW, _N), jnp.float32),
            pltpu.VMEM((_RPW, 128), jnp.float32),
            pltpu.VMEM((_N,), jnp.int32),
            pltpu.SemaphoreType.DMA,
        ],
    )


def _stage2_kernel(sim_tl_ref, sim_tr_ref, aux_ref, mem_ref, out_ref):
    sim_tl = sim_tl_ref[...]
    sim_tr = sim_tr_ref[...]
    a_term = aux_ref[0:1, :].T
    b_l = aux_ref[1:2, :]
    b_r = aux_ref[2:3, :]
    pos_dist = aux_ref[3:4, :].T
    contrastive = aux_ref[4:5, 0:1]

    mem = mem_ref[...] > 0
    member = jnp.any(mem, axis=0, keepdims=True)
    mem_l = member[:, 0:_H]
    mem_r = member[:, _H:_N]

    d2_l = a_term + b_l - 2.0 * sim_tl
    d2_r = a_term + b_r - 2.0 * sim_tr
    negd2 = jnp.minimum(
        jnp.min(jnp.where(mem_l, d2_l, _BIG), axis=1, keepdims=True),
        jnp.min(jnp.where(mem_r, d2_r, _BIG), axis=1, keepdims=True))
    neg_dist = jnp.sqrt(jnp.maximum(negd2, 0.0))

    triplet = jnp.sum(jnp.maximum(pos_dist - neg_dist + _MARGIN, 0.0),
                      keepdims=True) / _H
    out_ref[...] = triplet + _LAMBDA_C * contrastive


def kernel(img_emb, cap_emb, labels):
    hard, mb, sim_tl, sim_tr, aux = pl.pallas_call(
        _stage1_kernel,
        out_shape=[
            jax.ShapeDtypeStruct((_N, _N), jnp.float32),
            jax.ShapeDtypeStruct((_N, 128), jnp.float32),
            jax.ShapeDtypeStruct((_H, _H), jnp.float32),
            jax.ShapeDtypeStruct((_H, _H), jnp.float32),
            jax.ShapeDtypeStruct((8, _H), jnp.float32),
        ],
    )(img_emb.astype(jnp.float32), cap_emb.astype(jnp.float32))

    mem = _sc_call()(hard, mb)

    out = pl.pallas_call(
        _stage2_kernel,
        out_shape=jax.ShapeDtypeStruct((1, 1), jnp.float32),
    )(sim_tl, sim_tr, aux, mem.reshape(_NW, _N))
    return out[0, 0]


# speedup vs baseline: 4.3633x; 4.3633x over previous
import jax
import jax.numpy as jnp
from jax.experimental import pallas as pl
from jax.experimental.pallas import tpu as pltpu

_H = 256
_D = 512
_KB = 128
_NK = _D // _KB
_T = 0.6
_EPS = 1e-6
_MARGIN = 1.0
_LAMBDA_C = 0.5
_BIG = 3.4e38


def _dot_t(a, b):
    return jax.lax.dot_general(a, b, (((1,), (1,)), ((), ())),
                               preferred_element_type=jnp.float32)


def _rowsum(a, ones):
    return jax.lax.dot_general(a, ones, (((1,), (0,)), ((), ())),
                               preferred_element_type=jnp.float32)


def _loss_kernel(img_ref, cap_ref, out_ref,
                 gtl_ref, gtr_ref, gbb_ref, ssx_ref):
    k = pl.program_id(0)
    xi = img_ref[...]
    xj = cap_ref[...]
    ones = jnp.full((_KB, 1), 1.0, jnp.float32)

    gtl = _dot_t(xi, xi)
    gtr = _dot_t(xi, xj)
    gbb = _dot_t(xj, xj)
    ssx = jnp.concatenate(
        [_rowsum(xi * xi, ones), _rowsum(xj * xj, ones),
         _rowsum(xi, ones), _rowsum(xj, ones)], axis=1)

    @pl.when(k == 0)
    def _init():
        gtl_ref[...] = gtl
        gtr_ref[...] = gtr
        gbb_ref[...] = gbb
        ssx_ref[...] = ssx

    @pl.when(k > 0)
    def _acc():
        gtl_ref[...] += gtl
        gtr_ref[...] += gtr
        gbb_ref[...] += gbb
        ssx_ref[...] += ssx

    @pl.when(k == _NK - 1)
    def _epilogue():
        ss_i = ssx_ref[:, 0:1]
        ss_j = ssx_ref[:, 1:2]
        sx_i = ssx_ref[:, 2:3]
        sx_j = ssx_ref[:, 3:4]
        ri = 1.0 / jnp.maximum(jnp.sqrt(ss_i), 1e-12)
        rj = 1.0 / jnp.maximum(jnp.sqrt(ss_j), 1e-12)
        sim_tl = gtl_ref[...] * ri * ri.T
        sim_tr = gtr_ref[...] * ri * rj.T
        sim_bb = gbb_ref[...] * rj * rj.T

        i2 = jax.lax.broadcasted_iota(jnp.int32, (_H, _H), 0)
        j2 = jax.lax.broadcasted_iota(jnp.int32, (_H, _H), 1)
        diag = i2 == j2
        hard_tl = jnp.where(diag, 0.0, sim_tl)
        hard_tr = jnp.where(diag, 0.0, sim_tr)
        hard_bb = jnp.where(diag, 0.0, sim_bb)
        hard_bl = hard_tr.T

        m_top = jnp.maximum(jnp.max(hard_tl, axis=1, keepdims=True),
                            jnp.max(hard_tr, axis=1, keepdims=True))
        m_bot = jnp.maximum(jnp.max(hard_bl, axis=1, keepdims=True),
                            jnp.max(hard_bb, axis=1, keepdims=True))

        mem_l = (jnp.any(hard_tl == m_top, axis=0, keepdims=True)
                 | jnp.any(hard_bl == m_bot, axis=0, keepdims=True))
        mem_r = (jnp.any(hard_tr == m_top, axis=0, keepdims=True)
                 | jnp.any(hard_bb == m_bot, axis=0, keepdims=True))

        pos_sel = (((j2 == i2 + 128) & (i2 < 128))
                   | ((j2 == i2 - 128) & (i2 >= 128)))
        sum_pos = jnp.sum(jnp.where(pos_sel, sim_tl, 0.0), keepdims=True)
        contrastive = (jnp.sum(m_top, keepdims=True)
                       + jnp.sum(m_bot, keepdims=True)
                       - 2.0 * sum_pos) / (_T * _H)

        sq_i = ss_i * ri * ri
        sq_j = ss_j * rj * rj
        s_i = sx_i * ri
        s_j = sx_j * rj
        a_term = sq_i + 2.0 * _EPS * s_i + (_D * _EPS * _EPS)
        b_l = (sq_i - 2.0 * _EPS * s_i).T
        b_r = (sq_j - 2.0 * _EPS * s_j).T
        d2_l = a_term + b_l - 2.0 * sim_tl
        d2_r = a_term + b_r - 2.0 * sim_tr
        negd2 = jnp.minimum(
            jnp.min(jnp.where(mem_l, d2_l, _BIG), axis=1, keepdims=True),
            jnp.min(jnp.where(mem_r, d2_r, _BIG), axis=1, keepdims=True))
        neg_dist = jnp.sqrt(jnp.maximum(negd2, 0.0))

        dtr = jnp.sum(jnp.where(diag, sim_tr, 0.0), axis=1, keepdims=True)
        pd2 = (sq_i + sq_j - 2.0 * dtr
               + 2.0 * _EPS * (s_i - s_j) + (_D * _EPS * _EPS))
        pos_dist = jnp.sqrt(jnp.maximum(pd2, 0.0))

        triplet = jnp.sum(jnp.maximum(pos_dist - neg_dist + _MARGIN, 0.0),
                          keepdims=True) / _H
        out_ref[...] = triplet + _LAMBDA_C * contrastive


def kernel(img_emb, cap_emb, labels):
    out = pl.pallas_call(
        _loss_kernel,
        grid=(_NK,),
        in_specs=[
            pl.BlockSpec((_H, _KB), lambda k: (0, k)),
            pl.BlockSpec((_H, _KB), lambda k: (0, k)),
        ],
        out_specs=pl.BlockSpec((1, 1), lambda k: (0, 0)),
        out_shape=jax.ShapeDtypeStruct((1, 1), jnp.float32),
        scratch_shapes=[
            pltpu.VMEM((_H, _H), jnp.float32),
            pltpu.VMEM((_H, _H), jnp.float32),
            pltpu.VMEM((_H, _H), jnp.float32),
            pltpu.VMEM((_H, 4), jnp.float32),
        ],
    )(img_emb.astype(jnp.float32), cap_emb.astype(jnp.float32))
    return out[0, 0]


# speedup vs baseline: 8.3823x; 1.9211x over previous
import jax
import jax.numpy as jnp
from jax.experimental import pallas as pl

_H = 256
_D = 512
_T = 0.6
_EPS = 1e-6
_MARGIN = 1.0
_LAMBDA_C = 0.5
_BIG = 3.4e38


def _dot_t(a, b):
    return jax.lax.dot_general(a, b, (((1,), (1,)), ((), ())),
                               preferred_element_type=jnp.float32)


def _rowsum(a, ones):
    return jax.lax.dot_general(a, ones, (((1,), (0,)), ((), ())),
                               preferred_element_type=jnp.float32)


def _loss_kernel(img_ref, cap_ref, out_ref):
    xi = img_ref[...]
    xj = cap_ref[...]
    ones = jnp.full((_D, 1), 1.0, jnp.float32)

    ss_i = _rowsum(xi * xi, ones)
    ss_j = _rowsum(xj * xj, ones)
    sx_i = _rowsum(xi, ones)
    sx_j = _rowsum(xj, ones)
    ri = 1.0 / jnp.maximum(jnp.sqrt(ss_i), 1e-12)
    rj = 1.0 / jnp.maximum(jnp.sqrt(ss_j), 1e-12)
    zi = xi * ri
    zj = xj * rj

    sim_tl = _dot_t(zi, zi)
    sim_tr = _dot_t(zi, zj)
    sim_bb = _dot_t(zj, zj)

    i2 = jax.lax.broadcasted_iota(jnp.int32, (_H, _H), 0)
    j2 = jax.lax.broadcasted_iota(jnp.int32, (_H, _H), 1)
    diag = i2 == j2

    hard_tl = jnp.where(diag, 0.0, sim_tl)
    hard_tr = jnp.where(diag, 0.0, sim_tr)
    hard_bb = jnp.where(diag, 0.0, sim_bb)
    hard_bl = hard_tr.T

    m_top = jnp.maximum(jnp.max(hard_tl, axis=1, keepdims=True),
                        jnp.max(hard_tr, axis=1, keepdims=True))
    m_bot = jnp.maximum(jnp.max(hard_bl, axis=1, keepdims=True),
                        jnp.max(hard_bb, axis=1, keepdims=True))

    mem_l = (jnp.any(hard_tl == m_top, axis=0, keepdims=True)
             | jnp.any(hard_bl == m_bot, axis=0, keepdims=True))
    mem_r = (jnp.any(hard_tr == m_top, axis=0, keepdims=True)
             | jnp.any(hard_bb == m_bot, axis=0, keepdims=True))

    pos_sel = ((j2 == i2 + 128) & (i2 < 128)) | ((j2 == i2 - 128) & (i2 >= 128))
    sum_pos = jnp.sum(jnp.where(pos_sel, sim_tl, 0.0), keepdims=True)
    contrastive = (jnp.sum(m_top, keepdims=True) + jnp.sum(m_bot, keepdims=True)
                   - 2.0 * sum_pos) / (_T * _H)

    sq_i = ss_i * ri * ri
    sq_j = ss_j * rj * rj
    s_i = sx_i * ri
    s_j = sx_j * rj
    a_term = sq_i + 2.0 * _EPS * s_i + (_D * _EPS * _EPS)
    b_l = (sq_i - 2.0 * _EPS * s_i).T
    b_r = (sq_j - 2.0 * _EPS * s_j).T
    d2_l = a_term + b_l - 2.0 * sim_tl
    d2_r = a_term + b_r - 2.0 * sim_tr
    negd2 = jnp.minimum(
        jnp.min(jnp.where(mem_l, d2_l, _BIG), axis=1, keepdims=True),
        jnp.min(jnp.where(mem_r, d2_r, _BIG), axis=1, keepdims=True))
    neg_dist = jnp.sqrt(jnp.maximum(negd2, 0.0))

    dpos = zi - zj + _EPS
    pos_dist = jnp.sqrt(_rowsum(dpos * dpos, ones))

    triplet = jnp.sum(jnp.maximum(pos_dist - neg_dist + _MARGIN, 0.0),
                      keepdims=True) / _H
    out_ref[...] = triplet + _LAMBDA_C * contrastive


def kernel(img_emb, cap_emb, labels):
    out = pl.pallas_call(
        _loss_kernel,
        out_shape=jax.ShapeDtypeStruct((1, 1), jnp.float32),
    )(img_emb.astype(jnp.float32), cap_emb.astype(jnp.float32))
    return out[0, 0]
